# baseline (speedup 1.0000x reference)
import functools

import jax
import jax.numpy as jnp
from jax import lax
from jax.experimental import pallas as pl
from jax.experimental.pallas import tpu as pltpu
from jax.experimental.pallas import tpu_sc as plsc

DIM = 64
PK = 128
L = 16
CH = 128
NW = 32
ROWS = 4096 * 50
RPW = ROWS // NW
NCH = RPW // CH
GRP = CH // L
EPS = 1e-6


def _ln_chunk(rows_v, out_v, oidx_v, c, a2_v, b2_v):
    def group(g, carry):
        base = g * L
        rid = base + lax.iota(jnp.int32, L)
        idxv = oidx_v[c, pl.ds(base, L)]
        colbase = (idxv & 1) * DIM
        acc = jnp.zeros((L,), jnp.float32)
        acc2 = jnp.zeros((L,), jnp.float32)
        for d in range(DIM):
            v = plsc.load_gather(rows_v, [rid, colbase + d])
            acc = acc + v
            acc2 = acc2 + v * v
        mean = acc * (1.0 / DIM)
        var = (acc2 - acc * mean) * (1.0 / (DIM - 1))
        var = jnp.maximum(var, 0.0)
        y = plsc.bitcast(
            jnp.int32(0x5F3759DF) - (plsc.bitcast(var, jnp.int32) >> 1),
            jnp.float32,
        )
        for _ in range(3):
            y = y * (1.5 - 0.5 * var * y * y)
        inv = 1.0 / (var * y + EPS)
        orid = rid >> 1
        obase = (rid & 1) * DIM
        for d in range(DIM):
            dd = jnp.full((L,), d, jnp.int32)
            v = plsc.load_gather(rows_v, [rid, colbase + d])
            a2b = plsc.load_gather(a2_v, [dd])
            b2b = plsc.load_gather(b2_v, [dd])
            o = (v - mean) * inv * a2b + b2b
            plsc.store_scatter(out_v, [orid, obase + d], o)
        return carry

    lax.fori_loop(0, GRP, group, 0)


def _body(pidx_hbm, oidx_hbm, table_hbm, a2_hbm, b2_hbm, out_hbm,
          pidx_v, oidx_v, rows_v, out_v, a2_v, b2_v, sem):
    wid = lax.axis_index("s") * 2 + lax.axis_index("c")
    pltpu.sync_copy(pidx_hbm.at[wid], pidx_v)
    pltpu.sync_copy(oidx_hbm.at[wid], oidx_v)
    pltpu.sync_copy(a2_hbm, a2_v)
    pltpu.sync_copy(b2_hbm, b2_v)

    def chunk(c, carry):
        pltpu.async_copy(table_hbm.at[pidx_v.at[c]], rows_v, sem).wait()
        _ln_chunk(rows_v, out_v, oidx_v, c, a2_v, b2_v)
        obase = pl.multiple_of((wid * RPW + c * CH) // 2, CH // 2)
        pltpu.sync_copy(out_v, out_hbm.at[pl.ds(obase, CH // 2)])
        return carry

    lax.fori_loop(0, NCH, chunk, 0)


def kernel(inp, table, a_2, b_2):
    b, s = inp.shape
    flat = inp.reshape(NW, NCH, CH)
    pidx = flat >> 1
    tbl = table.reshape(-1, PK)
    mesh = plsc.VectorSubcoreMesh(core_axis_name="c", subcore_axis_name="s")
    run = functools.partial(
        pl.kernel,
        out_type=jax.ShapeDtypeStruct((ROWS // 2, PK), jnp.float32),
        mesh=mesh,
        compiler_params=pltpu.CompilerParams(needs_layout_passes=False),
        scratch_types=[
            pltpu.VMEM((NCH, CH), jnp.int32),
            pltpu.VMEM((NCH, CH), jnp.int32),
            pltpu.VMEM((CH, PK), jnp.float32),
            pltpu.VMEM((CH // 2, PK), jnp.float32),
            pltpu.VMEM((DIM,), jnp.float32),
            pltpu.VMEM((DIM,), jnp.float32),
            pltpu.SemaphoreType.DMA,
        ],
    )(_body)
    out = run(pidx, flat, tbl, a_2, b_2)
    return out.reshape(b, s, DIM)


# speedup vs baseline: 1.7375x; 1.7375x over previous
import functools

import jax
import jax.numpy as jnp
from jax import lax
from jax.experimental import pallas as pl
from jax.experimental.pallas import tpu as pltpu
from jax.experimental.pallas import tpu_sc as plsc

DIM = 64
L = 16
CH = 128
NW = 32
ROWS = 4096 * 50
RPW = ROWS // NW
NCH = RPW // CH
GRP = 16
EPS = 1e-6

_DNUMS = lax.GatherDimensionNumbers(
    offset_dims=(), collapsed_slice_dims=(0,), start_index_map=(0,))


def _bcast_lanes(x, perms):
    for p in perms:
        x = x + lax.gather(x, p, _DNUMS, (1,),
                           mode=lax.GatherScatterMode.PROMISE_IN_BOUNDS)
    return x


def _ln_row(rows_v, out_v, row, a2k, b2k, perms):
    v = [rows_v[row, pl.ds(k * L, L)] for k in range(DIM // L)]
    s = (v[0] + v[1]) + (v[2] + v[3])
    q = (v[0] * v[0] + v[1] * v[1]) + (v[2] * v[2] + v[3] * v[3])
    s = _bcast_lanes(s, perms)
    q = _bcast_lanes(q, perms)
    mean = s * (1.0 / DIM)
    var = (q - s * mean) * (1.0 / (DIM - 1))
    var = jnp.maximum(var, 0.0)
    y = plsc.bitcast(
        jnp.int32(0x5F3759DF) - (plsc.bitcast(var, jnp.int32) >> 1),
        jnp.float32,
    )
    for _ in range(3):
        y = y * (1.5 - 0.5 * var * y * y)
    inv = 1.0 / (var * y + EPS)
    for k in range(DIM // L):
        out_v[row, pl.ds(k * L, L)] = (v[k] - mean) * inv * a2k[k] + b2k[k]


def _body(inp_hbm, table_hbm, a2_hbm, b2_hbm, out_hbm,
          idx_v, rows_v, out_v, a2_v, b2_v, sem):
    wid = lax.axis_index("s") * 2 + lax.axis_index("c")
    pltpu.sync_copy(inp_hbm.at[wid], idx_v)
    pltpu.sync_copy(a2_hbm, a2_v)
    pltpu.sync_copy(b2_hbm, b2_v)
    a2k = [a2_v[pl.ds(k * L, L)] for k in range(DIM // L)]
    b2k = [b2_v[pl.ds(k * L, L)] for k in range(DIM // L)]
    iota = jnp.arange(L, dtype=jnp.int32)
    perms = [((iota ^ (1 << b))[:, None]) for b in range(4)]

    def chunk(c, carry):
        pltpu.async_copy(table_hbm.at[idx_v.at[c]], rows_v, sem).wait()

        def group(g, inner):
            base = g * GRP
            for r in range(GRP):
                _ln_row(rows_v, out_v, base + r, a2k, b2k, perms)
            return inner

        lax.fori_loop(0, CH // GRP, group, 0)
        pltpu.sync_copy(out_v, out_hbm.at[pl.ds(wid * RPW + c * CH, CH)])
        return carry

    lax.fori_loop(0, NCH, chunk, 0)


def kernel(inp, table, a_2, b_2):
    b, s = inp.shape
    inp_r = inp.reshape(NW, NCH, CH)
    mesh = plsc.VectorSubcoreMesh(core_axis_name="c", subcore_axis_name="s")
    run = functools.partial(
        pl.kernel,
        out_type=jax.ShapeDtypeStruct((ROWS, DIM), jnp.float32),
        mesh=mesh,
        compiler_params=pltpu.CompilerParams(
            needs_layout_passes=False, use_tc_tiling_on_sc=False),
        scratch_types=[
            pltpu.VMEM((NCH, CH), jnp.int32),
            pltpu.VMEM((CH, DIM), jnp.float32),
            pltpu.VMEM((CH, DIM), jnp.float32),
            pltpu.VMEM((DIM,), jnp.float32),
            pltpu.VMEM((DIM,), jnp.float32),
            pltpu.SemaphoreType.DMA,
        ],
    )(_body)
    out = run(inp_r, table, a_2, b_2)
    return out.reshape(b, s, DIM)
